# baseline (speedup 1.0000x reference)
import functools

import jax
import jax.numpy as jnp
import numpy as np
from jax import lax
from jax.experimental import pallas as pl
from jax.experimental.pallas import tpu as pltpu
from jax.experimental.pallas import tpu_sc as plsc

RES = 128
N_RAYS = 4096
N_PTS = 2097152
ROWS = N_PTS // 128
BR = 512
NC, NS, L = 2, 16, 16
NW = NC * NS
PTS_PER_W = N_PTS // NW
CHUNK = 8192
NCHUNK = PTS_PER_W // CHUNK
RSL = N_RAYS // NS


def _prep_body(pos_ref, alpha_ref, occs_ref, w_ref, table_ref, vidx_ref):
    a = alpha_ref[...]
    av = jnp.clip(a, 1e-06, 1.0 - 1e-06)
    ent = -av * jnp.log(av) - (1.0 - av) * jnp.log(1.0 - av)
    surf = occs_ref[...] > 0.8
    val = jnp.where(surf & (a <= 0.0), 0.0, ent)
    bits = lax.bitcast_convert_type(val, jnp.int32)
    table_ref[...] = jnp.where(surf, bits | jnp.int32(-(2**31)), bits).reshape(BR * 128)
    f = jnp.minimum(jnp.floor(pos_ref[...] * RES), RES - 1)
    vidx_f = jnp.dot(f.astype(jnp.bfloat16), w_ref[...],
                     preferred_element_type=jnp.float32)
    vidx_ref[...] = vidx_f.astype(jnp.int32).reshape(BR * 128)


_prep = pl.pallas_call(
    _prep_body,
    grid=(ROWS // BR,),
    in_specs=[
        pl.BlockSpec((BR, 3 * 128), lambda i: (i, 0)),
        pl.BlockSpec((BR, 128), lambda i: (i, 0)),
        pl.BlockSpec((BR, 128), lambda i: (i, 0)),
        pl.BlockSpec((3 * 128, 128), lambda i: (0, 0)),
    ],
    out_specs=[
        pl.BlockSpec((BR * 128,), lambda i: (i,)),
        pl.BlockSpec((BR * 128,), lambda i: (i,)),
    ],
    out_shape=[
        jax.ShapeDtypeStruct((N_PTS,), jnp.int32),
        jax.ShapeDtypeStruct((N_PTS,), jnp.int32),
    ],
)

_mesh = plsc.VectorSubcoreMesh(core_axis_name="c", subcore_axis_name="s")


@functools.partial(
    pl.kernel,
    mesh=_mesh,
    compiler_params=pltpu.CompilerParams(needs_layout_passes=False),
    out_type=(
        jax.ShapeDtypeStruct((N_PTS,), jnp.int32),
        jax.ShapeDtypeStruct((NC, N_RAYS), jnp.float32),
    ),
    scratch_types=[
        pltpu.VMEM((CHUNK,), jnp.int32),
        pltpu.VMEM((CHUNK,), jnp.int32),
        pltpu.VMEM((CHUNK,), jnp.int32),
        pltpu.VMEM((N_RAYS,), jnp.float32),
        pltpu.VMEM((RSL,), jnp.float32),
        pltpu.VMEM((RSL,), jnp.float32),
        pltpu.VMEM_SHARED((NS, N_RAYS), jnp.float32),
        pltpu.SemaphoreType.DMA,
    ],
)
def _gather_flags(vidx_hbm, table_hbm, ray_hbm, g_hbm, flags_hbm,
                  idx_v, g_v, ray_v, fl_v, acc_v, tmp_v, fl_sh, sem):
    c = lax.axis_index("c")
    s = lax.axis_index("s")
    wid = c * NS + s
    base = wid * PTS_PER_W

    zero16 = jnp.zeros((L,), jnp.float32)

    def zf(i, carry):
        fl_v[pl.ds(i * L, L)] = zero16
        return carry

    lax.fori_loop(0, N_RAYS // L, zf, 0)

    ones = jnp.ones((L,), jnp.float32)

    def chunk_body(k, carry):
        off = base + k * CHUNK
        pltpu.sync_copy(vidx_hbm.at[pl.ds(off, CHUNK)], idx_v)
        pltpu.async_copy(table_hbm.at[idx_v], g_v, sem).wait()
        pltpu.sync_copy(ray_hbm.at[pl.ds(off, CHUNK)], ray_v)
        pltpu.sync_copy(g_v, g_hbm.at[pl.ds(off, CHUNK)])

        def vec_body(j, carry2):
            gi = g_v[pl.ds(j * L, L)]
            rv = ray_v[pl.ds(j * L, L)]
            plsc.store_scatter(fl_v, [rv], ones, mask=gi < 0)
            return carry2

        lax.fori_loop(0, CHUNK // L, vec_body, 0)
        return carry

    lax.fori_loop(0, NCHUNK, chunk_body, 0)

    pltpu.sync_copy(fl_v, fl_sh.at[s])
    plsc.subcore_barrier()
    pltpu.sync_copy(fl_sh.at[0, pl.ds(s * RSL, RSL)], acc_v)

    def red_body(t, carry):
        pltpu.sync_copy(fl_sh.at[t, pl.ds(s * RSL, RSL)], tmp_v)

        def add_body(j, carry2):
            acc_v[pl.ds(j * L, L)] = acc_v[pl.ds(j * L, L)] + tmp_v[pl.ds(j * L, L)]
            return carry2

        lax.fori_loop(0, RSL // L, add_body, 0)
        return carry

    lax.fori_loop(1, NS, red_body, 0)
    pltpu.sync_copy(acc_v, flags_hbm.at[c, pl.ds(s * RSL, RSL)])


@functools.partial(
    pl.kernel,
    mesh=_mesh,
    compiler_params=pltpu.CompilerParams(needs_layout_passes=False),
    out_type=jax.ShapeDtypeStruct((N_PTS,), jnp.float32),
    scratch_types=[
        pltpu.VMEM((CHUNK,), jnp.int32),
        pltpu.VMEM((CHUNK,), jnp.int32),
        pltpu.VMEM((CHUNK,), jnp.float32),
        pltpu.VMEM((N_RAYS,), jnp.float32),
        pltpu.VMEM((N_RAYS,), jnp.float32),
    ],
)
def _finalize(g_hbm, ray_hbm, flags_hbm, out_hbm, g_v, ray_v, o_v, fl_v, fl2_v):
    c = lax.axis_index("c")
    s = lax.axis_index("s")
    wid = c * NS + s
    base = wid * PTS_PER_W

    pltpu.sync_copy(flags_hbm.at[0], fl_v)
    pltpu.sync_copy(flags_hbm.at[1], fl2_v)

    def add_body(j, carry):
        fl_v[pl.ds(j * L, L)] = fl_v[pl.ds(j * L, L)] + fl2_v[pl.ds(j * L, L)]
        return carry

    lax.fori_loop(0, N_RAYS // L, add_body, 0)

    mag_mask = jnp.full((L,), 0x7FFFFFFF, jnp.int32)
    zero16 = jnp.zeros((L,), jnp.float32)

    def chunk_body(k, carry):
        off = base + k * CHUNK
        pltpu.sync_copy(g_hbm.at[pl.ds(off, CHUNK)], g_v)
        pltpu.sync_copy(ray_hbm.at[pl.ds(off, CHUNK)], ray_v)

        def vec_body(j, carry2):
            gi = g_v[pl.ds(j * L, L)]
            rv = ray_v[pl.ds(j * L, L)]
            mag = plsc.bitcast(gi & mag_mask, jnp.float32)
            fr = plsc.load_gather(fl_v, [rv])
            keep = (gi < 0) | (fr <= 0.0)
            o_v[pl.ds(j * L, L)] = jnp.where(keep, mag, zero16)
            return carry2

        lax.fori_loop(0, CHUNK // L, vec_body, 0)
        pltpu.sync_copy(o_v, out_hbm.at[pl.ds(off, CHUNK)])
        return carry

    lax.fori_loop(0, NCHUNK, chunk_body, 0)


def _make_w():
    w = np.zeros((3 * 128, 128), np.float32)
    j = np.arange(128)
    w[3 * j, j] = RES * RES
    w[3 * j + 1, j] = RES
    w[3 * j + 2, j] = 1.0
    return w


_W = _make_w()


def kernel(positions, ray_indices, alpha, occs):
    pos2 = positions.reshape(ROWS, 3 * 128)
    alpha2 = alpha.reshape(ROWS, 128)
    occs2 = occs.reshape(ROWS, 128)
    table, vidx = _prep(pos2, alpha2, occs2, jnp.asarray(_W, jnp.bfloat16))
    g, flags = _gather_flags(vidx, table, ray_indices)
    return _finalize(g, ray_indices, flags)


# speedup vs baseline: 3.4344x; 3.4344x over previous
import functools

import jax
import jax.numpy as jnp
import numpy as np
from jax import lax
from jax.experimental import pallas as pl
from jax.experimental.pallas import tpu as pltpu
from jax.experimental.pallas import tpu_sc as plsc

RES = 128
N_RAYS = 4096
N_PTS = 2097152
ROWS = N_PTS // 128
BR = 512
BW = BR * 128
NC, NS, L = 2, 16, 16
NW = NC * NS
PTS_PER_W = N_PTS // NW
CHUNK = 4096
NCHUNK = PTS_PER_W // CHUNK
RSL = N_RAYS // NS
SCALE = 47268.0
INV_SCALE = 1.0 / SCALE
NPAIR = N_PTS // 2
SBLK = 16384
NBLK = 63
LIM = NBLK * SBLK
TAIL = NPAIR - LIM


def _prep_body(pos_ref, alpha_ref, occs_ref, ray_ref, table_ref,
               pidx_ref, rp_ref):
    a = alpha_ref[...]
    av = jnp.clip(a, 1e-06, 1.0 - 1e-06)
    ent = -av * jnp.log(av) - (1.0 - av) * jnp.log(1.0 - av)
    surf = occs_ref[...] > 0.8
    q = jnp.round(ent * SCALE)
    val = jnp.where(surf & (a <= 0.0), 0.0, q).astype(jnp.int32)
    bits = jnp.where(surf, val | 0x8000, val)
    table_ref[...] = bits.astype(jnp.int16).reshape(BW)
    fx = jnp.minimum(jnp.floor(pos_ref[0, :] * RES), RES - 1)
    fy = jnp.minimum(jnp.floor(pos_ref[1, :] * RES), RES - 1)
    fz = jnp.minimum(jnp.floor(pos_ref[2, :] * RES), RES - 1)
    vidx = (fx * (RES * RES) + fy * RES + fz).astype(jnp.int32)
    pidx = vidx >> 1
    pidx_ref[...] = jnp.minimum(pidx, LIM - 1)
    ovf = jnp.where(pidx >= LIM, ((pidx - LIM) << 13) | (1 << 27), 0)
    rp_ref[...] = ray_ref[...] | ((vidx & 1) << 12) | ovf


_prep = pl.pallas_call(
    _prep_body,
    grid=(ROWS // BR,),
    in_specs=[
        pl.BlockSpec((3, BW), lambda i: (0, i)),
        pl.BlockSpec((BR, 128), lambda i: (i, 0)),
        pl.BlockSpec((BR, 128), lambda i: (i, 0)),
        pl.BlockSpec((BW,), lambda i: (i,)),
    ],
    out_specs=[
        pl.BlockSpec((BW,), lambda i: (i,)),
        pl.BlockSpec((BW,), lambda i: (i,)),
        pl.BlockSpec((BW,), lambda i: (i,)),
    ],
    out_shape=[
        jax.ShapeDtypeStruct((N_PTS,), jnp.int16),
        jax.ShapeDtypeStruct((N_PTS,), jnp.int32),
        jax.ShapeDtypeStruct((N_PTS,), jnp.int32),
    ],
)

_mesh = plsc.VectorSubcoreMesh(core_axis_name="c", subcore_axis_name="s")


@functools.partial(
    pl.kernel,
    mesh=_mesh,
    compiler_params=pltpu.CompilerParams(needs_layout_passes=False),
    out_type=(
        jax.ShapeDtypeStruct((N_PTS,), jnp.int32),
        jax.ShapeDtypeStruct((NC, N_RAYS), jnp.float32),
    ),
    scratch_types=[
        pltpu.VMEM((CHUNK,), jnp.int32),
        pltpu.VMEM((CHUNK,), jnp.int32),
        pltpu.VMEM((CHUNK,), jnp.int32),
        pltpu.VMEM((CHUNK,), jnp.int32),
        pltpu.VMEM((CHUNK,), jnp.int32),
        pltpu.VMEM((CHUNK,), jnp.int32),
        pltpu.VMEM((CHUNK,), jnp.int32),
        pltpu.VMEM((CHUNK,), jnp.int32),
        pltpu.VMEM((TAIL,), jnp.int32),
        pltpu.VMEM((N_RAYS,), jnp.int32),
        pltpu.VMEM((RSL,), jnp.int32),
        pltpu.VMEM((RSL,), jnp.int32),
        pltpu.VMEM((RSL,), jnp.float32),
        pltpu.VMEM_SHARED((LIM,), jnp.int32),
        pltpu.SemaphoreType.DMA,
        pltpu.SemaphoreType.DMA,
        pltpu.SemaphoreType.DMA,
        pltpu.SemaphoreType.DMA,
        pltpu.SemaphoreType.DMA,
        pltpu.SemaphoreType.DMA,
    ],
)
def _gather_flags(pidx_hbm, table_hbm, rp_hbm, g_hbm, flags_hbm,
                  idx_v0, idx_v1, g_v0, g_v1, rp_v0, rp_v1, w_v0, w_v1,
                  tail_ts, fl_v, tmp_v, acc_v, accf_v, tab_sh,
                  s_idx0, s_idx1, s_rp0, s_rp1, s_g, s_w):
    c = lax.axis_index("c")
    s = lax.axis_index("s")
    wid = c * NS + s
    base = wid * PTS_PER_W
    idx_v = (idx_v0, idx_v1)
    g_v = (g_v0, g_v1)
    rp_v = (rp_v0, rp_v1)
    w_v = (w_v0, w_v1)
    s_idx = (s_idx0, s_idx1)
    s_rp = (s_rp0, s_rp1)

    h_idx = [None, None]
    h_rp = [None, None]
    h_idx[0] = pltpu.async_copy(pidx_hbm.at[pl.ds(base, CHUNK)],
                                idx_v[0], s_idx[0])
    h_rp[0] = pltpu.async_copy(rp_hbm.at[pl.ds(base, CHUNK)],
                               rp_v[0], s_rp[0])

    def stage_body(t, carry):
        blk = s + t * NS

        @pl.when(blk < NBLK)
        def _():
            pltpu.sync_copy(table_hbm.at[pl.ds(blk * SBLK, SBLK)],
                            tab_sh.at[pl.ds(blk * SBLK, SBLK)])

        return carry

    lax.fori_loop(0, (NBLK + NS - 1) // NS, stage_body, 0)
    pltpu.sync_copy(table_hbm.at[pl.ds(LIM, TAIL)], tail_ts)

    zero16 = jnp.zeros((L,), jnp.int32)

    def zf(i, carry):
        fl_v[pl.ds(i * L, L)] = zero16
        return carry

    lax.fori_loop(0, N_RAYS // L, zf, 0)
    plsc.subcore_barrier()

    ones = jnp.ones((L,), jnp.int32)
    par_bit = jnp.full((L,), 1 << 12, jnp.int32)
    ray_mask = jnp.full((L,), 0xFFF, jnp.int32)
    u16_mask = jnp.full((L,), 0xFFFF, jnp.int32)
    surf_bit = jnp.full((L,), 0x8000, jnp.int32)
    ovf_bit = jnp.full((L,), 1 << 27, jnp.int32)
    toff_mask = jnp.full((L,), 0x3FFF, jnp.int32)
    zi = jnp.zeros((L,), jnp.int32)

    pend_w = []
    for k in range(NCHUNK):
        b = k % 2
        off = base + k * CHUNK
        if len(pend_w) >= 2:
            pend_w.pop(0).wait()
        h_idx[b].wait()
        gather = pltpu.async_copy(tab_sh.at[idx_v[b]], g_v[b], s_g)
        if k + 1 < NCHUNK:
            off2 = base + (k + 1) * CHUNK
            h_idx[1 - b] = pltpu.async_copy(
                pidx_hbm.at[pl.ds(off2, CHUNK)], idx_v[1 - b], s_idx[1 - b])
            h_rp[1 - b] = pltpu.async_copy(
                rp_hbm.at[pl.ds(off2, CHUNK)], rp_v[1 - b], s_rp[1 - b])
        gather.wait()
        h_rp[b].wait()

        def vec_body(j, b=b):
            pair_s = g_v[b][pl.ds(j * L, L)]
            rpv = rp_v[b][pl.ds(j * L, L)]
            tpair = plsc.load_gather(tail_ts, [(rpv >> 13) & toff_mask])
            pair = jnp.where((rpv & ovf_bit) != zi, tpair, pair_s)
            odd = (rpv & par_bit) != zi
            sel = jnp.where(odd, pair >> 16, pair) & u16_mask
            w_v[b][pl.ds(j * L, L)] = sel
            rv = rpv & ray_mask
            plsc.store_scatter(fl_v, [rv], ones,
                               mask=(sel & surf_bit) != zi)

        plsc.parallel_loop(0, CHUNK // L, unroll=8)(vec_body)
        pend_w.append(
            pltpu.async_copy(w_v[b], g_hbm.at[pl.ds(off, CHUNK)], s_w))
    while pend_w:
        pend_w.pop(0).wait()

    xbase = c * (PTS_PER_W * NS)
    plsc.subcore_barrier()
    pltpu.sync_copy(fl_v, pidx_hbm.at[pl.ds(xbase + s * N_RAYS, N_RAYS)])
    plsc.subcore_barrier()
    pltpu.sync_copy(pidx_hbm.at[pl.ds(xbase + s * RSL, RSL)], acc_v)

    def red_body(t, carry):
        pltpu.sync_copy(
            pidx_hbm.at[pl.ds(xbase + t * N_RAYS + s * RSL, RSL)], tmp_v)

        def add_body(j, carry2):
            acc_v[pl.ds(j * L, L)] = (acc_v[pl.ds(j * L, L)]
                                      + tmp_v[pl.ds(j * L, L)])
            return carry2

        lax.fori_loop(0, RSL // L, add_body, 0)
        return carry

    lax.fori_loop(1, NS, red_body, 0)

    def conv_body(j, carry):
        accf_v[pl.ds(j * L, L)] = acc_v[pl.ds(j * L, L)].astype(jnp.float32)
        return carry

    lax.fori_loop(0, RSL // L, conv_body, 0)
    pltpu.sync_copy(accf_v, flags_hbm.at[c, pl.ds(s * RSL, RSL)])


@functools.partial(
    pl.kernel,
    mesh=_mesh,
    compiler_params=pltpu.CompilerParams(needs_layout_passes=False),
    out_type=jax.ShapeDtypeStruct((N_PTS,), jnp.float32),
    scratch_types=[
        pltpu.VMEM((CHUNK,), jnp.int32),
        pltpu.VMEM((CHUNK,), jnp.int32),
        pltpu.VMEM((CHUNK,), jnp.int32),
        pltpu.VMEM((CHUNK,), jnp.int32),
        pltpu.VMEM((CHUNK,), jnp.float32),
        pltpu.VMEM((CHUNK,), jnp.float32),
        pltpu.VMEM((N_RAYS,), jnp.float32),
        pltpu.VMEM((N_RAYS,), jnp.float32),
        pltpu.SemaphoreType.DMA,
        pltpu.SemaphoreType.DMA,
        pltpu.SemaphoreType.DMA,
        pltpu.SemaphoreType.DMA,
        pltpu.SemaphoreType.DMA,
    ],
)
def _finalize(g_hbm, rp_hbm, flags_hbm, out_hbm, g_v0, g_v1, rp_v0, rp_v1,
              o_v0, o_v1, fl_v, fl2_v, s_g0, s_g1, s_rp0, s_rp1, s_o):
    c = lax.axis_index("c")
    s = lax.axis_index("s")
    wid = c * NS + s
    base = wid * PTS_PER_W
    g_v = (g_v0, g_v1)
    rp_v = (rp_v0, rp_v1)
    o_v = (o_v0, o_v1)
    s_g = (s_g0, s_g1)
    s_rp = (s_rp0, s_rp1)

    h_g = [None, None]
    h_rp = [None, None]
    h_g[0] = pltpu.async_copy(g_hbm.at[pl.ds(base, CHUNK)], g_v[0], s_g[0])
    h_rp[0] = pltpu.async_copy(rp_hbm.at[pl.ds(base, CHUNK)],
                               rp_v[0], s_rp[0])

    pltpu.sync_copy(flags_hbm.at[0], fl_v)
    pltpu.sync_copy(flags_hbm.at[1], fl2_v)

    def add_body(j, carry):
        fl_v[pl.ds(j * L, L)] = fl_v[pl.ds(j * L, L)] + fl2_v[pl.ds(j * L, L)]
        return carry

    lax.fori_loop(0, N_RAYS // L, add_body, 0, unroll=4)

    mag_mask = jnp.full((L,), 0x7FFF, jnp.int32)
    surf_bit = jnp.full((L,), 0x8000, jnp.int32)
    ray_mask = jnp.full((L,), 0xFFF, jnp.int32)
    zi = jnp.zeros((L,), jnp.int32)
    zerof = jnp.zeros((L,), jnp.float32)
    inv_scale = jnp.full((L,), INV_SCALE, jnp.float32)

    pend_o = []
    for k in range(NCHUNK):
        b = k % 2
        off = base + k * CHUNK
        if k + 1 < NCHUNK:
            off2 = base + (k + 1) * CHUNK
            h_g[1 - b] = pltpu.async_copy(
                g_hbm.at[pl.ds(off2, CHUNK)], g_v[1 - b], s_g[1 - b])
            h_rp[1 - b] = pltpu.async_copy(
                rp_hbm.at[pl.ds(off2, CHUNK)], rp_v[1 - b], s_rp[1 - b])
        if len(pend_o) >= 2:
            pend_o.pop(0).wait()
        h_g[b].wait()
        h_rp[b].wait()

        def vec_body(j, b=b):
            v = g_v[b][pl.ds(j * L, L)]
            rv = rp_v[b][pl.ds(j * L, L)] & ray_mask
            mag = (v & mag_mask).astype(jnp.float32) * inv_scale
            fr = plsc.load_gather(fl_v, [rv])
            keep = ((v & surf_bit) != zi) | (fr <= 0.0)
            o_v[b][pl.ds(j * L, L)] = jnp.where(keep, mag, zerof)

        plsc.parallel_loop(0, CHUNK // L, unroll=8)(vec_body)
        pend_o.append(
            pltpu.async_copy(o_v[b], out_hbm.at[pl.ds(off, CHUNK)], s_o))
    while pend_o:
        pend_o.pop(0).wait()


def kernel(positions, ray_indices, alpha, occs):
    pos_t = positions.T
    alpha2 = alpha.reshape(ROWS, 128)
    occs2 = occs.reshape(ROWS, 128)
    table, pidx, rp = _prep(pos_t, alpha2, occs2, ray_indices)
    table32 = lax.bitcast_convert_type(table.reshape(NPAIR, 2), jnp.int32)
    g, flags = _gather_flags(pidx, table32, rp)
    return _finalize(g, rp, flags)


# speedup vs baseline: 22.0596x; 6.4232x over previous
import functools

import jax
import jax.numpy as jnp
import numpy as np
from jax import lax
from jax.experimental import pallas as pl
from jax.experimental.pallas import tpu as pltpu
from jax.experimental.pallas import tpu_sc as plsc

RES = 128
N_RAYS = 4096
N_PTS = 2097152
ROWS = N_PTS // 128
BR = 512
BW = BR * 128
NC, NS, L = 2, 16, 16
NW = NC * NS
PTS_PER_W = N_PTS // NW
CHUNK = 4096
NCHUNK = PTS_PER_W // CHUNK
RSL = N_RAYS // NS
SCALE = 47268.0
INV_SCALE = 1.0 / SCALE
NPAIR = N_PTS // 2
SBLK = 16384
NBLK = 63
LIM = NBLK * SBLK
TAIL = NPAIR - LIM


def _bits16(a, o):
    av = jnp.clip(a, 1e-06, 1.0 - 1e-06)
    ent = -av * jnp.log(av) - (1.0 - av) * jnp.log(1.0 - av)
    surf = o > 0.8
    q = jnp.round(ent * SCALE)
    val = jnp.where(surf & (a <= 0.0), 0.0, q).astype(jnp.int32)
    return jnp.where(surf, val | 0x8000, val)


def _prep_body(pos_ref, alo_ref, ahi_ref, olo_ref, ohi_ref, ray_ref,
               table_ref, pidx_ref, rp_ref):
    blo = _bits16(alo_ref[...], olo_ref[...])
    bhi = _bits16(ahi_ref[...], ohi_ref[...])
    table_ref[...] = blo | (bhi << 16)
    fx = jnp.minimum(jnp.floor(pos_ref[0, :] * RES), RES - 1)
    fy = jnp.minimum(jnp.floor(pos_ref[1, :] * RES), RES - 1)
    fz = jnp.minimum(jnp.floor(pos_ref[2, :] * RES), RES - 1)
    vidx = (fx * (RES * RES) + fy * RES + fz).astype(jnp.int32)
    pidx = vidx & (NPAIR - 1)
    pidx_ref[...] = jnp.minimum(pidx, LIM - 1)
    ovf = jnp.where(pidx >= LIM, ((pidx - LIM) << 13) | (1 << 27), 0)
    rp_ref[...] = ray_ref[...] | ((vidx >> 20) << 12) | ovf


_HROWS = ROWS // 2
_PBR = _HROWS // (ROWS // BR)

_prep = pl.pallas_call(
    _prep_body,
    grid=(ROWS // BR,),
    in_specs=[
        pl.BlockSpec((3, BW), lambda i: (0, i)),
        pl.BlockSpec((_PBR, 128), lambda i: (i, 0)),
        pl.BlockSpec((_PBR, 128), lambda i: (i + ROWS // BR, 0)),
        pl.BlockSpec((_PBR, 128), lambda i: (i, 0)),
        pl.BlockSpec((_PBR, 128), lambda i: (i + ROWS // BR, 0)),
        pl.BlockSpec((BW,), lambda i: (i,)),
    ],
    out_specs=[
        pl.BlockSpec((_PBR, 128), lambda i: (i, 0)),
        pl.BlockSpec((BW,), lambda i: (i,)),
        pl.BlockSpec((BW,), lambda i: (i,)),
    ],
    out_shape=[
        jax.ShapeDtypeStruct((_HROWS, 128), jnp.int32),
        jax.ShapeDtypeStruct((N_PTS,), jnp.int32),
        jax.ShapeDtypeStruct((N_PTS,), jnp.int32),
    ],
)

_mesh = plsc.VectorSubcoreMesh(core_axis_name="c", subcore_axis_name="s")


@functools.partial(
    pl.kernel,
    mesh=_mesh,
    compiler_params=pltpu.CompilerParams(needs_layout_passes=False),
    out_type=(
        jax.ShapeDtypeStruct((N_PTS,), jnp.int32),
        jax.ShapeDtypeStruct((NC, N_RAYS), jnp.float32),
    ),
    scratch_types=[
        pltpu.VMEM((CHUNK,), jnp.int32),
        pltpu.VMEM((CHUNK,), jnp.int32),
        pltpu.VMEM((CHUNK,), jnp.int32),
        pltpu.VMEM((CHUNK,), jnp.int32),
        pltpu.VMEM((CHUNK,), jnp.int32),
        pltpu.VMEM((CHUNK,), jnp.int32),
        pltpu.VMEM((CHUNK,), jnp.int32),
        pltpu.VMEM((CHUNK,), jnp.int32),
        pltpu.VMEM((TAIL,), jnp.int32),
        pltpu.VMEM((N_RAYS,), jnp.int32),
        pltpu.VMEM((RSL,), jnp.int32),
        pltpu.VMEM((RSL,), jnp.int32),
        pltpu.VMEM((RSL,), jnp.float32),
        pltpu.VMEM_SHARED((LIM,), jnp.int32),
        pltpu.SemaphoreType.DMA,
        pltpu.SemaphoreType.DMA,
        pltpu.SemaphoreType.DMA,
        pltpu.SemaphoreType.DMA,
        pltpu.SemaphoreType.DMA,
        pltpu.SemaphoreType.DMA,
    ],
)
def _gather_flags(pidx_hbm, table_hbm, rp_hbm, g_hbm, flags_hbm,
                  idx_v0, idx_v1, g_v0, g_v1, rp_v0, rp_v1, w_v0, w_v1,
                  tail_ts, fl_v, tmp_v, acc_v, accf_v, tab_sh,
                  s_idx0, s_idx1, s_rp0, s_rp1, s_g, s_w):
    c = lax.axis_index("c")
    s = lax.axis_index("s")
    wid = c * NS + s
    base = wid * PTS_PER_W
    idx_v = (idx_v0, idx_v1)
    g_v = (g_v0, g_v1)
    rp_v = (rp_v0, rp_v1)
    w_v = (w_v0, w_v1)
    s_idx = (s_idx0, s_idx1)
    s_rp = (s_rp0, s_rp1)

    h_idx = [None, None]
    h_rp = [None, None]
    h_idx[0] = pltpu.async_copy(pidx_hbm.at[pl.ds(base, CHUNK)],
                                idx_v[0], s_idx[0])
    h_rp[0] = pltpu.async_copy(rp_hbm.at[pl.ds(base, CHUNK)],
                               rp_v[0], s_rp[0])

    def stage_body(t, carry):
        blk = s + t * NS

        @pl.when(blk < NBLK)
        def _():
            pltpu.sync_copy(table_hbm.at[pl.ds(blk * SBLK, SBLK)],
                            tab_sh.at[pl.ds(blk * SBLK, SBLK)])

        return carry

    lax.fori_loop(0, (NBLK + NS - 1) // NS, stage_body, 0)
    pltpu.sync_copy(table_hbm.at[pl.ds(LIM, TAIL)], tail_ts)

    zero16 = jnp.zeros((L,), jnp.int32)

    def zf(i, carry):
        fl_v[pl.ds(i * L, L)] = zero16
        return carry

    lax.fori_loop(0, N_RAYS // L, zf, 0)
    plsc.subcore_barrier()

    ones = jnp.ones((L,), jnp.int32)
    par_bit = jnp.full((L,), 1 << 12, jnp.int32)
    ray_mask = jnp.full((L,), 0xFFF, jnp.int32)
    u16_mask = jnp.full((L,), 0xFFFF, jnp.int32)
    surf_bit = jnp.full((L,), 0x8000, jnp.int32)
    ovf_bit = jnp.full((L,), 1 << 27, jnp.int32)
    toff_mask = jnp.full((L,), 0x3FFF, jnp.int32)
    zi = jnp.zeros((L,), jnp.int32)

    pend_w = []
    for k in range(NCHUNK):
        b = k % 2
        off = base + k * CHUNK
        if len(pend_w) >= 2:
            pend_w.pop(0).wait()
        h_idx[b].wait()
        gather = pltpu.async_copy(tab_sh.at[idx_v[b]], g_v[b], s_g)
        if k + 1 < NCHUNK:
            off2 = base + (k + 1) * CHUNK
            h_idx[1 - b] = pltpu.async_copy(
                pidx_hbm.at[pl.ds(off2, CHUNK)], idx_v[1 - b], s_idx[1 - b])
            h_rp[1 - b] = pltpu.async_copy(
                rp_hbm.at[pl.ds(off2, CHUNK)], rp_v[1 - b], s_rp[1 - b])
        gather.wait()
        h_rp[b].wait()

        def vec_body(j, b=b):
            pair_s = g_v[b][pl.ds(j * L, L)]
            rpv = rp_v[b][pl.ds(j * L, L)]
            tpair = plsc.load_gather(tail_ts, [(rpv >> 13) & toff_mask])
            pair = jnp.where((rpv & ovf_bit) != zi, tpair, pair_s)
            odd = (rpv & par_bit) != zi
            sel = jnp.where(odd, pair >> 16, pair) & u16_mask
            w_v[b][pl.ds(j * L, L)] = sel
            rv = rpv & ray_mask
            plsc.store_scatter(fl_v, [rv], ones,
                               mask=(sel & surf_bit) != zi)

        plsc.parallel_loop(0, CHUNK // L, unroll=8)(vec_body)
        pend_w.append(
            pltpu.async_copy(w_v[b], g_hbm.at[pl.ds(off, CHUNK)], s_w))
    while pend_w:
        pend_w.pop(0).wait()

    xbase = c * (PTS_PER_W * NS)
    plsc.subcore_barrier()
    pltpu.sync_copy(fl_v, pidx_hbm.at[pl.ds(xbase + s * N_RAYS, N_RAYS)])
    plsc.subcore_barrier()
    pltpu.sync_copy(pidx_hbm.at[pl.ds(xbase + s * RSL, RSL)], acc_v)

    def red_body(t, carry):
        pltpu.sync_copy(
            pidx_hbm.at[pl.ds(xbase + t * N_RAYS + s * RSL, RSL)], tmp_v)

        def add_body(j, carry2):
            acc_v[pl.ds(j * L, L)] = (acc_v[pl.ds(j * L, L)]
                                      + tmp_v[pl.ds(j * L, L)])
            return carry2

        lax.fori_loop(0, RSL // L, add_body, 0)
        return carry

    lax.fori_loop(1, NS, red_body, 0)

    def conv_body(j, carry):
        accf_v[pl.ds(j * L, L)] = acc_v[pl.ds(j * L, L)].astype(jnp.float32)
        return carry

    lax.fori_loop(0, RSL // L, conv_body, 0)
    pltpu.sync_copy(accf_v, flags_hbm.at[c, pl.ds(s * RSL, RSL)])


@functools.partial(
    pl.kernel,
    mesh=_mesh,
    compiler_params=pltpu.CompilerParams(needs_layout_passes=False),
    out_type=jax.ShapeDtypeStruct((N_PTS,), jnp.float32),
    scratch_types=[
        pltpu.VMEM((CHUNK,), jnp.int32),
        pltpu.VMEM((CHUNK,), jnp.int32),
        pltpu.VMEM((CHUNK,), jnp.int32),
        pltpu.VMEM((CHUNK,), jnp.int32),
        pltpu.VMEM((CHUNK,), jnp.float32),
        pltpu.VMEM((CHUNK,), jnp.float32),
        pltpu.VMEM((N_RAYS,), jnp.float32),
        pltpu.VMEM((N_RAYS,), jnp.float32),
        pltpu.SemaphoreType.DMA,
        pltpu.SemaphoreType.DMA,
        pltpu.SemaphoreType.DMA,
        pltpu.SemaphoreType.DMA,
        pltpu.SemaphoreType.DMA,
    ],
)
def _finalize(g_hbm, rp_hbm, flags_hbm, out_hbm, g_v0, g_v1, rp_v0, rp_v1,
              o_v0, o_v1, fl_v, fl2_v, s_g0, s_g1, s_rp0, s_rp1, s_o):
    c = lax.axis_index("c")
    s = lax.axis_index("s")
    wid = c * NS + s
    base = wid * PTS_PER_W
    g_v = (g_v0, g_v1)
    rp_v = (rp_v0, rp_v1)
    o_v = (o_v0, o_v1)
    s_g = (s_g0, s_g1)
    s_rp = (s_rp0, s_rp1)

    h_g = [None, None]
    h_rp = [None, None]
    h_g[0] = pltpu.async_copy(g_hbm.at[pl.ds(base, CHUNK)], g_v[0], s_g[0])
    h_rp[0] = pltpu.async_copy(rp_hbm.at[pl.ds(base, CHUNK)],
                               rp_v[0], s_rp[0])

    pltpu.sync_copy(flags_hbm.at[0], fl_v)
    pltpu.sync_copy(flags_hbm.at[1], fl2_v)

    def add_body(j, carry):
        fl_v[pl.ds(j * L, L)] = fl_v[pl.ds(j * L, L)] + fl2_v[pl.ds(j * L, L)]
        return carry

    lax.fori_loop(0, N_RAYS // L, add_body, 0, unroll=4)

    mag_mask = jnp.full((L,), 0x7FFF, jnp.int32)
    surf_bit = jnp.full((L,), 0x8000, jnp.int32)
    ray_mask = jnp.full((L,), 0xFFF, jnp.int32)
    zi = jnp.zeros((L,), jnp.int32)
    zerof = jnp.zeros((L,), jnp.float32)
    inv_scale = jnp.full((L,), INV_SCALE, jnp.float32)

    pend_o = []
    for k in range(NCHUNK):
        b = k % 2
        off = base + k * CHUNK
        if k + 1 < NCHUNK:
            off2 = base + (k + 1) * CHUNK
            h_g[1 - b] = pltpu.async_copy(
                g_hbm.at[pl.ds(off2, CHUNK)], g_v[1 - b], s_g[1 - b])
            h_rp[1 - b] = pltpu.async_copy(
                rp_hbm.at[pl.ds(off2, CHUNK)], rp_v[1 - b], s_rp[1 - b])
        if len(pend_o) >= 2:
            pend_o.pop(0).wait()
        h_g[b].wait()
        h_rp[b].wait()

        def vec_body(j, b=b):
            v = g_v[b][pl.ds(j * L, L)]
            rv = rp_v[b][pl.ds(j * L, L)] & ray_mask
            mag = (v & mag_mask).astype(jnp.float32) * inv_scale
            fr = plsc.load_gather(fl_v, [rv])
            keep = ((v & surf_bit) != zi) | (fr <= 0.0)
            o_v[b][pl.ds(j * L, L)] = jnp.where(keep, mag, zerof)

        plsc.parallel_loop(0, CHUNK // L, unroll=8)(vec_body)
        pend_o.append(
            pltpu.async_copy(o_v[b], out_hbm.at[pl.ds(off, CHUNK)], s_o))
    while pend_o:
        pend_o.pop(0).wait()


def kernel(positions, ray_indices, alpha, occs):
    pos_t = positions.T
    alpha2 = alpha.reshape(ROWS, 128)
    occs2 = occs.reshape(ROWS, 128)
    table32, pidx, rp = _prep(pos_t, alpha2, alpha2, occs2, occs2,
                              ray_indices)
    g, flags = _gather_flags(pidx, table32.reshape(-1), rp)
    return _finalize(g, rp, flags)
